# baseline (speedup 1.0000x reference)
import jax
import jax.numpy as jnp
from jax.experimental import pallas as pl
from jax.experimental.pallas import tpu as pltpu

B, C_HALF, H, W = 32, 192, 64, 64
C_TOTAL = 2 * C_HALF
SUB, LANE = 32, 128


def _half_body(a1, a2, o, sel, x1_ref, x2_ref, out_ref):
    j = pl.program_id(0)

    @pl.when(sel[j] == 0)
    def _():
        out_ref[...] = x1_ref[...]

    @pl.when(sel[j] == 1)
    def _():
        out_ref[...] = x2_ref[...]


def _fill_forward(vals, mask):
    n = vals.shape[0]
    steps = jnp.arange(n, dtype=jnp.int32)
    marked = jnp.where(mask, steps, -1)
    last = jax.lax.cummax(marked)
    first = jnp.argmax(mask).astype(jnp.int32)
    idx = jnp.where(last >= 0, last, first)
    return jnp.where(jnp.any(mask), vals[idx], 0).astype(jnp.int32)


def _shuffle_half(x1r, x2r, srcs):
    order = jnp.argsort(srcs).astype(jnp.int32)
    ssort = jnp.sort(srcs).astype(jnp.int32)
    from_x2 = ssort >= C_HALF
    a1 = _fill_forward(ssort, jnp.logical_not(from_x2))
    a2 = _fill_forward(ssort - C_HALF, from_x2)
    sel = from_x2.astype(jnp.int32)

    block = (B, 1, SUB, LANE)
    grid_spec = pltpu.PrefetchScalarGridSpec(
        num_scalar_prefetch=4,
        grid=(C_HALF,),
        in_specs=[
            pl.BlockSpec(block, lambda j, a1, a2, o, s: (0, a1[j], 0, 0)),
            pl.BlockSpec(block, lambda j, a1, a2, o, s: (0, a2[j], 0, 0)),
        ],
        out_specs=pl.BlockSpec(block, lambda j, a1, a2, o, s: (0, o[j], 0, 0)),
    )
    return pl.pallas_call(
        _half_body,
        grid_spec=grid_spec,
        out_shape=jax.ShapeDtypeStruct((B, C_HALF, SUB, LANE), jnp.float32),
        compiler_params=pltpu.CompilerParams(dimension_semantics=("parallel",)),
    )(a1, a2, order, sel, x1r, x2r)


def kernel(x1, x2, sldj_x, fwd_idxs):
    x1r = x1.reshape(B, C_HALF, SUB, LANE)
    x2r = x2.reshape(B, C_HALF, SUB, LANE)
    out1 = _shuffle_half(x1r, x2r, fwd_idxs[:C_HALF])
    out2 = _shuffle_half(x1r, x2r, fwd_idxs[C_HALF:])
    return (
        out1.reshape(B, C_HALF, H, W),
        out2.reshape(B, C_HALF, H, W),
        sldj_x,
    )
